# baseline (speedup 1.0000x reference)
import jax
import jax.numpy as jnp
from jax.experimental import pallas as pl
from jax.experimental.pallas import tpu as pltpu

_ROWS = 4096
_COLS = 8192
_N_CHUNKS = 8
_CHUNK = _ROWS // _N_CHUNKS


def _dma_copy(x_ref, o_ref, sems):
    for i in range(_N_CHUNKS):
        pltpu.make_async_copy(
            x_ref.at[pl.ds(i * _CHUNK, _CHUNK), :],
            o_ref.at[pl.ds(i * _CHUNK, _CHUNK), :],
            sems.at[i],
        ).start()
    for i in range(_N_CHUNKS):
        pltpu.make_async_copy(
            x_ref.at[pl.ds(i * _CHUNK, _CHUNK), :],
            o_ref.at[pl.ds(i * _CHUNK, _CHUNK), :],
            sems.at[i],
        ).wait()


def kernel(x):
    return pl.pallas_call(
        _dma_copy,
        in_specs=[pl.BlockSpec(memory_space=pl.ANY)],
        out_specs=pl.BlockSpec(memory_space=pl.ANY),
        out_shape=jax.ShapeDtypeStruct((_ROWS, _COLS), x.dtype),
        scratch_shapes=[pltpu.SemaphoreType.DMA((_N_CHUNKS,))],
    )(x)
